# baseline (speedup 1.0000x reference)
import functools

import jax
import jax.numpy as jnp
from jax import lax
from jax.experimental import pallas as pl
from jax.experimental.pallas import tpu as pltpu
from jax.experimental.pallas import tpu_sc as plsc

N = 10000
E = 320000
D = 128
G = 128
EPS = 1e-5

NC = 2
NS = 16
NW = NC * NS
EPW = E // NW
K = 96
C = 105
EPWP = C * K
NP = 10112
RPS = NP // NS

NB = 10
BR = N // NB


def _make_sc_agg(W: int):
  mesh = plsc.VectorSubcoreMesh(core_axis_name="c", subcore_axis_name="s")

  def body(table_hbm, src_hbm, dst_hbm, out_hbm,
           src_v, dst_v, rows0, rows1, accum, g0, g1, s0, s1):
    cid = lax.axis_index("c")
    sid = lax.axis_index("s")
    wid = sid * NC + cid

    pltpu.sync_copy(src_hbm.at[wid], src_v)
    pltpu.sync_copy(dst_hbm.at[wid], dst_v)

    def zrow(r, carry):
      for l in range(W // 16):
        rows1[r, pl.ds(l * 16, 16)] = jnp.zeros((16,), jnp.float32)
      return carry
    lax.fori_loop(0, K, zrow, 0)
    for q in range(RPS // K):
      pltpu.sync_copy(rows1, accum.at[pl.ds(sid * RPS + q * K, K)])
    if RPS % K:
      pltpu.sync_copy(rows1.at[pl.ds(0, RPS % K)],
                      accum.at[pl.ds(sid * RPS + (RPS // K) * K, RPS % K)])
    plsc.subcore_barrier()

    def gather_start(j, rows, sem):
      pltpu.async_copy(table_hbm.at[src_v.at[pl.ds(j * K, K)]], rows, sem)

    def gather_wait(j, rows, sem):
      pltpu.make_async_copy(table_hbm.at[src_v.at[pl.ds(j * K, K)]], rows,
                            sem).wait()

    def scatter_start(j, rows, sem):
      pltpu.async_copy(rows, accum.at[dst_v.at[j]], sem, add=True)

    def scatter_wait(j, rows, sem):
      pltpu.make_async_copy(rows, accum.at[dst_v.at[j]], sem).wait()

    gather_start(0, rows0, g0)
    gather_start(1, rows1, g1)

    def pair(jj, carry):
      j0 = 2 * jj
      j1 = j0 + 1
      gather_wait(j0, rows0, g0)
      scatter_start(j0, rows0, s0)
      gather_wait(j1, rows1, g1)
      scatter_wait(j0, rows0, s0)

      @pl.when(j0 + 2 < C)
      def _():
        gather_start(j0 + 2, rows0, g0)

      scatter_start(j1, rows1, s1)
      scatter_wait(j1, rows1, s1)

      @pl.when(j1 + 2 < C)
      def _():
        gather_start(j1 + 2, rows1, g1)

      return carry
    lax.fori_loop(0, C // 2, pair, 0)
    if C % 2:
      j = C - 1
      gather_wait(j, rows0, g0)
      scatter_start(j, rows0, s0)
      scatter_wait(j, rows0, s0)
    plsc.subcore_barrier()

    pltpu.sync_copy(accum.at[pl.ds(sid * RPS, RPS)],
                    out_hbm.at[cid, pl.ds(sid * RPS, RPS)])

  return pl.kernel(
      body,
      out_type=jax.ShapeDtypeStruct((NC, NP, W), jnp.float32),
      mesh=mesh,
      scratch_types=[
          pltpu.VMEM((EPWP,), jnp.int32),
          pltpu.VMEM((C, K), jnp.int32),
          pltpu.VMEM((K, W), jnp.float32),
          pltpu.VMEM((K, W), jnp.float32),
          pltpu.VMEM_SHARED((NP, W), jnp.float32),
          pltpu.SemaphoreType.DMA,
          pltpu.SemaphoreType.DMA,
          pltpu.SemaphoreType.DMA,
          pltpu.SemaphoreType.DMA,
      ],
  )


_sc_agg_feat = _make_sc_agg(D)


def _sc_cnt_body(dst_hbm, out_hbm, dst_v, acc_v):
  cid = lax.axis_index("c")
  sid = lax.axis_index("s")
  wid = sid * NC + cid

  def zrow(i, carry):
    acc_v[pl.ds(i * 16, 16)] = jnp.zeros((16,), jnp.float32)
    return carry
  lax.fori_loop(0, NP // 16, zrow, 0)

  pltpu.sync_copy(dst_hbm.at[wid], dst_v)

  ones16 = jnp.full((16,), 1.0, jnp.float32)

  def step(i, carry):
    idx = dst_v[pl.ds(i * 16, 16)]
    plsc.addupdate_scatter(acc_v, [idx], ones16)
    return carry
  lax.fori_loop(0, EPW // 16, step, 0)

  pltpu.sync_copy(acc_v, out_hbm.at[wid])


_sc_cnt = pl.kernel(
    _sc_cnt_body,
    out_type=jax.ShapeDtypeStruct((NW, NP), jnp.float32),
    mesh=plsc.VectorSubcoreMesh(core_axis_name="c", subcore_axis_name="s"),
    scratch_types=[
        pltpu.VMEM((EPW,), jnp.int32),
        pltpu.VMEM((NP,), jnp.float32),
    ],
    compiler_params=pltpu.CompilerParams(needs_layout_passes=False),
)


def _make_tc_layer_body(final: bool):
  def body(*refs):
    if final:
      (part_ref, cntp_ref, h_ref, Wl_ref, bl_ref, Wr_ref, gamma_ref, beta_ref,
       batch_ref, fcW1_ref, fcb1_ref, fcW2_ref, fcb2_ref,
       out_ref, head_ref, z_s, stats_s, pooled_s) = refs
    else:
      (part_ref, cntp_ref, h_ref, Wl_ref, bl_ref, Wr_ref, gamma_ref, beta_ref,
       out_ref, z_s, stats_s) = refs
    p = pl.program_id(0)
    i = pl.program_id(1)

    @pl.when(p == 0)
    def _():
      agg = part_ref[0] + part_ref[1]
      cnt = jnp.sum(cntp_ref[...], axis=1, keepdims=True)
      mean = agg / jnp.maximum(cnt, 1.0)
      z = (lax.dot_general(mean, Wl_ref[...], (((1,), (1,)), ((), ())),
                           preferred_element_type=jnp.float32)
           + lax.dot_general(h_ref[...], Wr_ref[...], (((1,), (1,)), ((), ())),
                             preferred_element_type=jnp.float32)
           + bl_ref[...])
      z_s[pl.ds(i * BR, BR), :] = z

      @pl.when(i == 0)
      def _():
        stats_s[...] = jnp.zeros_like(stats_s)

      stats_s[0:1, :] += jnp.sum(z, axis=0, keepdims=True)
      stats_s[1:2, :] += jnp.sum(z * z, axis=0, keepdims=True)

    @pl.when(p == 1)
    def _():
      mu = stats_s[0:1, :] / N
      var = stats_s[1:2, :] / N - mu * mu
      scale = gamma_ref[...] * lax.rsqrt(var + EPS)
      z = z_s[pl.ds(i * BR, BR), :]
      h_new = jnp.maximum((z - mu) * scale + beta_ref[...], 0.0)
      out_ref[...] = h_new

      if final:
        @pl.when(i == 0)
        def _():
          pooled_s[...] = jnp.zeros_like(pooled_s)

        gids = lax.broadcasted_iota(jnp.int32, (1, G), 1)
        oh = (batch_ref[...] == gids).astype(jnp.float32)
        pooled_s[...] += lax.dot_general(
            oh, h_new, (((0,), (0,)), ((), ())),
            preferred_element_type=jnp.float32)

        @pl.when(i == NB - 1)
        def _():
          hid = lax.dot_general(pooled_s[...], fcW1_ref[...],
                                (((1,), (1,)), ((), ())),
                                preferred_element_type=jnp.float32
                                ) + fcb1_ref[...]
          hid = jnp.maximum(hid, 0.0)
          head_ref[...] = lax.dot_general(hid, fcW2_ref[...],
                                          (((1,), (1,)), ((), ())),
                                          preferred_element_type=jnp.float32
                                          ) + fcb2_ref[...]
  return body


def _make_tc_layer(final: bool):
  in_specs = [
      pl.BlockSpec((NC, BR, D), lambda p, i: (0, i * (1 - p), 0)),
      pl.BlockSpec((BR, NW), lambda p, i: (i * (1 - p), 0)),
      pl.BlockSpec((BR, D), lambda p, i: (i * (1 - p), 0)),
      pl.BlockSpec((D, D), lambda p, i: (0, 0)),
      pl.BlockSpec((1, D), lambda p, i: (0, 0)),
      pl.BlockSpec((D, D), lambda p, i: (0, 0)),
      pl.BlockSpec((1, D), lambda p, i: (0, 0)),
      pl.BlockSpec((1, D), lambda p, i: (0, 0)),
  ]
  out_specs = [pl.BlockSpec((BR, D), lambda p, i: (i * p, 0))]
  out_shape = [jax.ShapeDtypeStruct((N, D), jnp.float32)]
  scratch = [
      pltpu.VMEM((N, D), jnp.float32),
      pltpu.VMEM((2, D), jnp.float32),
  ]
  if final:
    in_specs += [
        pl.BlockSpec((BR, 1), lambda p, i: (i * p, 0)),
        pl.BlockSpec((G // 2, D), lambda p, i: (0, 0)),
        pl.BlockSpec((G, G // 2), lambda p, i: (0, 0)),
        pl.BlockSpec((G, G // 2), lambda p, i: (0, 0)),
        pl.BlockSpec((G, G), lambda p, i: (0, 0)),
    ]
    out_specs += [pl.BlockSpec((G, G), lambda p, i: (0, 0))]
    out_shape += [jax.ShapeDtypeStruct((G, G), jnp.float32)]
    scratch += [pltpu.VMEM((G, D), jnp.float32)]
  return pl.pallas_call(
      _make_tc_layer_body(final),
      grid=(2, NB),
      in_specs=in_specs,
      out_specs=out_specs,
      out_shape=out_shape,
      scratch_shapes=scratch,
  )


_tc_layer_mid = _make_tc_layer(False)
_tc_layer_fin = _make_tc_layer(True)


def kernel(x, edge_index, batch,
           Wl0, bl0, Wr0, gamma0, beta0,
           Wl1, bl1, Wr1, gamma1, beta1,
           Wl2, bl2, Wr2, gamma2, beta2,
           fcW1, fcb1, fcW2, fcb2):
  pad = EPWP - EPW
  src2 = jnp.pad(edge_index[0].reshape(NW, EPW), ((0, 0), (0, pad)))
  dstp = jnp.pad(edge_index[1].reshape(NW, EPW), ((0, 0), (0, pad)),
                 constant_values=NP - 1)
  dst3 = dstp.reshape(NW, C, K)
  dst2 = edge_index[1].reshape(NW, EPW)
  batch2 = batch.reshape(N, 1)

  cntp = _sc_cnt(dst2).T

  fcb1b = jnp.broadcast_to(fcb1.reshape(1, G // 2), (G, G // 2))
  fcW2p = jnp.zeros((G, G // 2), jnp.float32).at[0, :].set(fcW2[0])
  fcb2p = jnp.zeros((G, G), jnp.float32).at[:, 0].set(fcb2[0])

  h = x
  for li, (Wl, bl, Wr, gamma, beta) in enumerate((
      (Wl0, bl0, Wr0, gamma0, beta0),
      (Wl1, bl1, Wr1, gamma1, beta1),
      (Wl2, bl2, Wr2, gamma2, beta2),
  )):
    part = _sc_agg_feat(h, src2, dst3)
    args = (part, cntp, h, Wl, bl.reshape(1, D), Wr,
            gamma.reshape(1, D), beta.reshape(1, D))
    if li < 2:
      (h,) = _tc_layer_mid(*args)
    else:
      _, head = _tc_layer_fin(*args, batch2, fcW1, fcb1b, fcW2p, fcb2p)
  return head[:, :1]


# speedup vs baseline: 1.5640x; 1.5640x over previous
import functools

import jax
import jax.numpy as jnp
from jax import lax
from jax.experimental import pallas as pl
from jax.experimental.pallas import tpu as pltpu
from jax.experimental.pallas import tpu_sc as plsc

N = 10000
E = 320000
D = 128
G = 128
EPS = 1e-5

NC = 2
NS = 16
NW = NC * NS
EPW = E // NW
K = 80
C = EPW // K
NP = 10112
RPS = NP // NS

NB = 10
BR = N // NB


def _make_sc_agg(W: int):
  mesh = plsc.VectorSubcoreMesh(core_axis_name="c", subcore_axis_name="s")

  def body(table_hbm, src_hbm, dst_hbm, out_hbm,
           src_v, dst_v, rows0, rows1, accum, g0, g1, s0, s1):
    cid = lax.axis_index("c")
    sid = lax.axis_index("s")
    wid = sid * NC + cid

    pltpu.sync_copy(src_hbm.at[wid], src_v)
    pltpu.sync_copy(dst_hbm.at[wid], dst_v)

    def zrow(r, carry):
      for l in range(W // 16):
        rows1[r, pl.ds(l * 16, 16)] = jnp.zeros((16,), jnp.float32)
      return carry
    lax.fori_loop(0, K, zrow, 0)
    for q in range(RPS // K):
      pltpu.sync_copy(rows1, accum.at[pl.ds(sid * RPS + q * K, K)])
    if RPS % K:
      pltpu.sync_copy(rows1.at[pl.ds(0, RPS % K)],
                      accum.at[pl.ds(sid * RPS + (RPS // K) * K, RPS % K)])
    plsc.subcore_barrier()

    def gather_start(j, rows, sem):
      pltpu.async_copy(table_hbm.at[src_v.at[pl.ds(j * K, K)]], rows, sem)

    def gather_wait(j, rows, sem):
      pltpu.make_async_copy(table_hbm.at[src_v.at[pl.ds(j * K, K)]], rows,
                            sem).wait()

    def scatter_start(j, rows, sem):
      pltpu.async_copy(rows, accum.at[dst_v.at[j]], sem, add=True)

    def scatter_wait(j, rows, sem):
      pltpu.make_async_copy(rows, accum.at[dst_v.at[j]], sem).wait()

    gather_start(0, rows0, g0)
    gather_start(1, rows1, g1)

    def pair(jj, carry):
      j0 = 2 * jj
      j1 = j0 + 1
      gather_wait(j0, rows0, g0)
      scatter_start(j0, rows0, s0)
      gather_wait(j1, rows1, g1)
      scatter_wait(j0, rows0, s0)

      @pl.when(j0 + 2 < C)
      def _():
        gather_start(j0 + 2, rows0, g0)

      scatter_start(j1, rows1, s1)
      scatter_wait(j1, rows1, s1)

      @pl.when(j1 + 2 < C)
      def _():
        gather_start(j1 + 2, rows1, g1)

      return carry
    lax.fori_loop(0, C // 2, pair, 0)
    if C % 2:
      j = C - 1
      gather_wait(j, rows0, g0)
      scatter_start(j, rows0, s0)
      scatter_wait(j, rows0, s0)
    plsc.subcore_barrier()

    pltpu.sync_copy(accum.at[pl.ds(sid * RPS, RPS)],
                    out_hbm.at[cid, pl.ds(sid * RPS, RPS)])

  return pl.kernel(
      body,
      out_type=jax.ShapeDtypeStruct((NC, NP, W), jnp.float32),
      mesh=mesh,
      scratch_types=[
          pltpu.VMEM((EPW,), jnp.int32),
          pltpu.VMEM((C, K), jnp.int32),
          pltpu.VMEM((K, W), jnp.float32),
          pltpu.VMEM((K, W), jnp.float32),
          pltpu.VMEM_SHARED((NP, W), jnp.float32),
          pltpu.SemaphoreType.DMA,
          pltpu.SemaphoreType.DMA,
          pltpu.SemaphoreType.DMA,
          pltpu.SemaphoreType.DMA,
      ],
  )


_sc_agg_feat = _make_sc_agg(D)


def _sc_cnt_body(dst_hbm, out_hbm, dst_v, acc_v):
  cid = lax.axis_index("c")
  sid = lax.axis_index("s")
  wid = sid * NC + cid

  def zrow(i, carry):
    acc_v[pl.ds(i * 16, 16)] = jnp.zeros((16,), jnp.float32)
    return carry
  lax.fori_loop(0, NP // 16, zrow, 0)

  pltpu.sync_copy(dst_hbm.at[wid], dst_v)

  ones16 = jnp.full((16,), 1.0, jnp.float32)

  def step(i, carry):
    idx = dst_v[pl.ds(i * 16, 16)]
    plsc.addupdate_scatter(acc_v, [idx], ones16)
    return carry
  lax.fori_loop(0, EPW // 16, step, 0)

  pltpu.sync_copy(acc_v, out_hbm.at[wid])


_sc_cnt = pl.kernel(
    _sc_cnt_body,
    out_type=jax.ShapeDtypeStruct((NW, NP), jnp.float32),
    mesh=plsc.VectorSubcoreMesh(core_axis_name="c", subcore_axis_name="s"),
    scratch_types=[
        pltpu.VMEM((EPW,), jnp.int32),
        pltpu.VMEM((NP,), jnp.float32),
    ],
    compiler_params=pltpu.CompilerParams(needs_layout_passes=False),
)


def _make_tc_layer_body(final: bool):
  def body(*refs):
    if final:
      (part_ref, cntp_ref, h_ref, Wl_ref, bl_ref, Wr_ref, gamma_ref, beta_ref,
       batch_ref, fcW1_ref, fcb1_ref, fcW2_ref, fcb2_ref,
       out_ref, head_ref, z_s, stats_s, pooled_s) = refs
    else:
      (part_ref, cntp_ref, h_ref, Wl_ref, bl_ref, Wr_ref, gamma_ref, beta_ref,
       out_ref, z_s, stats_s) = refs
    p = pl.program_id(0)
    i = pl.program_id(1)

    @pl.when(p == 0)
    def _():
      agg = part_ref[0] + part_ref[1]
      cnt = jnp.sum(cntp_ref[...], axis=1, keepdims=True)
      mean = agg / jnp.maximum(cnt, 1.0)
      z = (lax.dot_general(mean, Wl_ref[...], (((1,), (1,)), ((), ())),
                           preferred_element_type=jnp.float32)
           + lax.dot_general(h_ref[...], Wr_ref[...], (((1,), (1,)), ((), ())),
                             preferred_element_type=jnp.float32)
           + bl_ref[...])
      z_s[pl.ds(i * BR, BR), :] = z

      @pl.when(i == 0)
      def _():
        stats_s[...] = jnp.zeros_like(stats_s)

      stats_s[0:1, :] += jnp.sum(z, axis=0, keepdims=True)
      stats_s[1:2, :] += jnp.sum(z * z, axis=0, keepdims=True)

    @pl.when(p == 1)
    def _():
      mu = stats_s[0:1, :] / N
      var = stats_s[1:2, :] / N - mu * mu
      scale = gamma_ref[...] * lax.rsqrt(var + EPS)
      z = z_s[pl.ds(i * BR, BR), :]
      h_new = jnp.maximum((z - mu) * scale + beta_ref[...], 0.0)
      out_ref[...] = h_new

      if final:
        @pl.when(i == 0)
        def _():
          pooled_s[...] = jnp.zeros_like(pooled_s)

        gids = lax.broadcasted_iota(jnp.int32, (1, G), 1)
        oh = (batch_ref[...] == gids).astype(jnp.float32)
        pooled_s[...] += lax.dot_general(
            oh, h_new, (((0,), (0,)), ((), ())),
            preferred_element_type=jnp.float32)

        @pl.when(i == NB - 1)
        def _():
          hid = lax.dot_general(pooled_s[...], fcW1_ref[...],
                                (((1,), (1,)), ((), ())),
                                preferred_element_type=jnp.float32
                                ) + fcb1_ref[...]
          hid = jnp.maximum(hid, 0.0)
          head_ref[...] = lax.dot_general(hid, fcW2_ref[...],
                                          (((1,), (1,)), ((), ())),
                                          preferred_element_type=jnp.float32
                                          ) + fcb2_ref[...]
  return body


def _make_tc_layer(final: bool):
  in_specs = [
      pl.BlockSpec((NC, BR, D), lambda p, i: (0, i * (1 - p), 0)),
      pl.BlockSpec((BR, NW), lambda p, i: (i * (1 - p), 0)),
      pl.BlockSpec((BR, D), lambda p, i: (i * (1 - p), 0)),
      pl.BlockSpec((D, D), lambda p, i: (0, 0)),
      pl.BlockSpec((1, D), lambda p, i: (0, 0)),
      pl.BlockSpec((D, D), lambda p, i: (0, 0)),
      pl.BlockSpec((1, D), lambda p, i: (0, 0)),
      pl.BlockSpec((1, D), lambda p, i: (0, 0)),
  ]
  out_specs = [pl.BlockSpec((BR, D), lambda p, i: (i * p, 0))]
  out_shape = [jax.ShapeDtypeStruct((N, D), jnp.float32)]
  scratch = [
      pltpu.VMEM((N, D), jnp.float32),
      pltpu.VMEM((2, D), jnp.float32),
  ]
  if final:
    in_specs += [
        pl.BlockSpec((BR, 1), lambda p, i: (i * p, 0)),
        pl.BlockSpec((G // 2, D), lambda p, i: (0, 0)),
        pl.BlockSpec((G, G // 2), lambda p, i: (0, 0)),
        pl.BlockSpec((G, G // 2), lambda p, i: (0, 0)),
        pl.BlockSpec((G, G), lambda p, i: (0, 0)),
    ]
    out_specs += [pl.BlockSpec((G, G), lambda p, i: (0, 0))]
    out_shape += [jax.ShapeDtypeStruct((G, G), jnp.float32)]
    scratch += [pltpu.VMEM((G, D), jnp.float32)]
  return pl.pallas_call(
      _make_tc_layer_body(final),
      grid=(2, NB),
      in_specs=in_specs,
      out_specs=out_specs,
      out_shape=out_shape,
      scratch_shapes=scratch,
  )


_tc_layer_mid = _make_tc_layer(False)
_tc_layer_fin = _make_tc_layer(True)


def kernel(x, edge_index, batch,
           Wl0, bl0, Wr0, gamma0, beta0,
           Wl1, bl1, Wr1, gamma1, beta1,
           Wl2, bl2, Wr2, gamma2, beta2,
           fcW1, fcb1, fcW2, fcb2):
  src2 = edge_index[0].reshape(NW, EPW)
  dst3 = edge_index[1].reshape(NW, C, K)
  dst2 = edge_index[1].reshape(NW, EPW)
  batch2 = batch.reshape(N, 1)

  cntp = _sc_cnt(dst2).T

  fcb1b = jnp.broadcast_to(fcb1.reshape(1, G // 2), (G, G // 2))
  fcW2p = jnp.zeros((G, G // 2), jnp.float32).at[0, :].set(fcW2[0])
  fcb2p = jnp.zeros((G, G), jnp.float32).at[:, 0].set(fcb2[0])

  h = x
  for li, (Wl, bl, Wr, gamma, beta) in enumerate((
      (Wl0, bl0, Wr0, gamma0, beta0),
      (Wl1, bl1, Wr1, gamma1, beta1),
      (Wl2, bl2, Wr2, gamma2, beta2),
  )):
    part = _sc_agg_feat(h, src2, dst3)
    args = (part, cntp, h, Wl, bl.reshape(1, D), Wr,
            gamma.reshape(1, D), beta.reshape(1, D))
    if li < 2:
      (h,) = _tc_layer_mid(*args)
    else:
      _, head = _tc_layer_fin(*args, batch2, fcW1, fcb1b, fcW2p, fcb2p)
  return head[:, :1]
